# baseline (speedup 1.0000x reference)
import functools

import jax
import jax.numpy as jnp
from jax import lax
from jax.experimental import pallas as pl
from jax.experimental.pallas import tpu as pltpu
from jax.experimental.pallas import tpu_sc as plsc

_EPS = 1e-5
_NCORES = 2
_NSUB = 16
_CHUNK = 80
_ZROWS = 128
_NBUF = 4


def _pad_rows(n):
    q = _NSUB * _ZROWS
    return ((n + q - 1) // q) * q


def _fill_const(ref, rows, d, val):
    def body(i, carry):
        for l in range(d // 16):
            ref[i, pl.ds(l * 16, 16)] = jnp.full((16,), val, jnp.float32)
        return carry

    lax.fori_loop(0, rows, body, None)


def _deg_call(ei4, n):
    nw, steps, _, c = ei4.shape
    n_pad = _pad_rows(n)
    mesh = plsc.VectorSubcoreMesh(core_axis_name="c", subcore_axis_name="s")

    @functools.partial(
        pl.kernel,
        mesh=mesh,
        out_type=jax.ShapeDtypeStruct((nw, n_pad), jnp.float32),
        compiler_params=pltpu.CompilerParams(needs_layout_passes=False),
        scratch_types=[
            pltpu.VMEM((steps, 2, c), jnp.int32),
            pltpu.VMEM((n_pad,), jnp.float32),
        ],
    )
    def deg_k(ei_hbm, out_hbm, eidx, hist):
        cid = lax.axis_index("c")
        sid = lax.axis_index("s")
        wid = cid * _NSUB + sid

        def z(i, carry):
            hist[pl.ds(i * 16, 16)] = jnp.zeros((16,), jnp.float32)
            return carry

        lax.fori_loop(0, n_pad // 16, z, None)
        pltpu.sync_copy(ei_hbm.at[wid], eidx)
        ones16 = jnp.full((16,), 1.0, jnp.float32)

        def row(r, carry):
            for l in range(c // 16):
                v = eidx[r, 1, pl.ds(l * 16, 16)]
                plsc.addupdate_scatter(hist, [v], ones16)
            return carry

        lax.fori_loop(0, steps, row, None)
        pltpu.sync_copy(hist, out_hbm.at[wid])

    return deg_k(ei4)


_NIDX = 8


def _agg_call(hs, ei4):
    n, d = hs.shape
    nw, steps, _, c = ei4.shape
    epw = steps * c
    n_pad = _pad_rows(n)
    rpt = n_pad // _NSUB
    mesh = plsc.VectorSubcoreMesh(core_axis_name="c", subcore_axis_name="s")

    @functools.partial(
        pl.kernel,
        mesh=mesh,
        out_type=jax.ShapeDtypeStruct((_NCORES, n_pad, d), jnp.float32),
        scratch_types=(
            [pltpu.VMEM((c, d), jnp.float32)] * _NBUF
            + [pltpu.VMEM((2, c), jnp.int32)] * _NIDX
            + [pltpu.VMEM_SHARED((n_pad, d), jnp.float32),
               pltpu.SemaphoreType.DMA,
               pltpu.SemaphoreType.DMA,
               pltpu.SemaphoreType.DMA]
        ),
    )
    def agg_k(hs_hbm, ei_hbm, out_hbm, *refs):
        bufs = refs[:_NBUF]
        idxs = refs[_NBUF:_NBUF + _NIDX]
        acc_sh, isem, gsem, ssem = refs[_NBUF + _NIDX:]
        cid = lax.axis_index("c")
        sid = lax.axis_index("s")
        wid = cid * _NSUB + sid

        _fill_const(bufs[0], c, d, 0.0)
        t0 = sid * rpt
        for k in range(rpt // c):
            pltpu.sync_copy(bufs[0], acc_sh.at[pl.ds(t0 + k * c, c)])
        plsc.subcore_barrier()

        def idx_load(j, slot):
            pltpu.async_copy(ei_hbm.at[wid, j], idxs[slot], isem)

        def idx_drain(slot):
            pltpu.make_async_copy(ei_hbm.at[0, 0], idxs[slot], isem).wait()

        def g_drain(b):
            pltpu.make_async_copy(hs_hbm.at[pl.ds(0, c)], bufs[b],
                                  gsem).wait()

        def s_drain(b):
            pltpu.make_async_copy(hs_hbm.at[pl.ds(0, c)], bufs[b],
                                  ssem).wait()

        for j in range(4):
            idx_load(j, j)
        for j in range(2):
            idx_drain(j)
            pltpu.async_copy(hs_hbm.at[idxs[j].at[0]], bufs[j], gsem)

        def chunk_body(j, b, islot):
            static = isinstance(j, int)
            g_drain(b)
            pltpu.async_copy(bufs[b], acc_sh.at[idxs[islot].at[1]], ssem,
                             add=True)

            def drain_prev_scatter():
                s_drain((b + 2) % _NBUF)

            def next_gather():
                idx_drain((islot + 2) % _NIDX)
                pltpu.async_copy(hs_hbm.at[idxs[(islot + 2) % _NIDX].at[0]],
                                 bufs[(b + 2) % _NBUF], gsem)

            def next_idx():
                idx_load(j + 4, (islot + 4) % _NIDX)

            if static:
                if j >= 2:
                    drain_prev_scatter()
                if j + 2 < steps:
                    next_gather()
                if j + 4 < steps:
                    next_idx()
            else:
                pl.when(j >= 2)(drain_prev_scatter)
                pl.when(j + 2 < steps)(next_gather)
                pl.when(j + 4 < steps)(next_idx)

        def blk(jj, carry):
            j0 = jj * _NIDX
            for r in range(_NIDX):
                chunk_body(j0 + r, r % _NBUF, r)
            return carry

        lax.fori_loop(0, steps // _NIDX, blk, None)
        for r in range(steps % _NIDX):
            j = (steps // _NIDX) * _NIDX + r
            chunk_body(j, j % _NBUF, j % _NIDX)
        for r in range(2):
            s_drain(0)
        plsc.subcore_barrier()

        for k in range(rpt // c):
            r0 = t0 + k * c
            pltpu.sync_copy(acc_sh.at[pl.ds(r0, c)], bufs[0])
            pltpu.sync_copy(bufs[0], out_hbm.at[cid, pl.ds(r0, c)])

    return agg_k(hs, ei4)


_BN = 2000


def _tc1_call(x, degp):
    n, d = x.shape
    nw = _NCORES * _NSUB
    bn = 2048

    def tc1(x_ref, deg_ref, xs_ref, dinv_ref):
        cnt = jnp.dot(deg_ref[...].T, jnp.ones((nw, 1), jnp.float32),
                      preferred_element_type=jnp.float32)
        dinv = lax.rsqrt(cnt + 1.0)
        xs_ref[...] = x_ref[...] * dinv
        dinv_ref[...] = dinv

    return pl.pallas_call(
        tc1,
        grid=(pl.cdiv(n, bn),),
        in_specs=[
            pl.BlockSpec((bn, d), lambda i: (i, 0)),
            pl.BlockSpec((nw, bn), lambda i: (0, i)),
        ],
        out_specs=[
            pl.BlockSpec((bn, d), lambda i: (i, 0)),
            pl.BlockSpec((bn, 1), lambda i: (i, 0)),
        ],
        out_shape=[
            jax.ShapeDtypeStruct((n, d), jnp.float32),
            jax.ShapeDtypeStruct((n, 1), jnp.float32),
        ],
    )(x, degp)


def _tc2_call(acc1, xs, dinv, x, w1, b1, g1, be1):
    n, h = xs.shape
    bn_scale = 1.0 / (1.0 + _EPS) ** 0.5

    def tc2(acc_ref, xs_ref, dinv_ref, x_ref, w_ref, b_ref, g_ref, be_ref,
            y_ref, ys_ref):
        a = acc_ref[...]
        dinv = dinv_ref[...]
        agg = (a[0] + a[1] + xs_ref[...]) * dinv
        pre = jnp.dot(agg, w_ref[...], preferred_element_type=jnp.float32) \
            + b_ref[...]
        bn = pre * (g_ref[...] * bn_scale) + be_ref[...]
        y = jnp.maximum(bn, 0.0) + x_ref[...]
        y_ref[...] = y
        ys_ref[...] = y * dinv

    return pl.pallas_call(
        tc2,
        grid=(n // _BN,),
        in_specs=[
            pl.BlockSpec((_NCORES, _BN, h), lambda i: (0, i, 0)),
            pl.BlockSpec((_BN, h), lambda i: (i, 0)),
            pl.BlockSpec((_BN, 1), lambda i: (i, 0)),
            pl.BlockSpec((_BN, h), lambda i: (i, 0)),
            pl.BlockSpec((h, h), lambda i: (0, 0)),
            pl.BlockSpec((1, h), lambda i: (0, 0)),
            pl.BlockSpec((1, h), lambda i: (0, 0)),
            pl.BlockSpec((1, h), lambda i: (0, 0)),
        ],
        out_specs=[
            pl.BlockSpec((_BN, h), lambda i: (i, 0)),
            pl.BlockSpec((_BN, h), lambda i: (i, 0)),
        ],
        out_shape=[
            jax.ShapeDtypeStruct((n, h), jnp.float32),
            jax.ShapeDtypeStruct((n, h), jnp.float32),
        ],
    )(acc1, xs, dinv, x, w1, b1, g1, be1)


def _tc3_call(acc2, ys1, dinv, y1, w2, b2, g2, be2, wo1, bo1, wo2, bo2):
    n, h = ys1.shape
    hh = wo1.shape[1]
    nc = wo2.shape[1]
    bn_scale = 1.0 / (1.0 + _EPS) ** 0.5

    def tc3(acc_ref, ys_ref, dinv_ref, y1_ref, w_ref, b_ref, g_ref, be_ref,
            wo1_ref, bo1_ref, wo2_ref, bo2_ref, o_ref):
        a = acc_ref[...]
        dinv = dinv_ref[...]
        agg = (a[0] + a[1] + ys_ref[...]) * dinv
        pre = jnp.dot(agg, w_ref[...], preferred_element_type=jnp.float32) \
            + b_ref[...]
        bn = pre * (g_ref[...] * bn_scale) + be_ref[...]
        y2 = jnp.maximum(bn, 0.0) + y1_ref[...]
        t = jnp.dot(y2, wo1_ref[...], preferred_element_type=jnp.float32)
        t = jnp.maximum(t + bo1_ref[...], 0.0)
        o = jnp.dot(t, wo2_ref[...], preferred_element_type=jnp.float32)
        o_ref[...] = o + bo2_ref[...]

    return pl.pallas_call(
        tc3,
        grid=(n // _BN,),
        in_specs=[
            pl.BlockSpec((_NCORES, _BN, h), lambda i: (0, i, 0)),
            pl.BlockSpec((_BN, h), lambda i: (i, 0)),
            pl.BlockSpec((_BN, 1), lambda i: (i, 0)),
            pl.BlockSpec((_BN, h), lambda i: (i, 0)),
            pl.BlockSpec((h, h), lambda i: (0, 0)),
            pl.BlockSpec((1, h), lambda i: (0, 0)),
            pl.BlockSpec((1, h), lambda i: (0, 0)),
            pl.BlockSpec((1, h), lambda i: (0, 0)),
            pl.BlockSpec((h, hh), lambda i: (0, 0)),
            pl.BlockSpec((1, hh), lambda i: (0, 0)),
            pl.BlockSpec((hh, nc), lambda i: (0, 0)),
            pl.BlockSpec((1, nc), lambda i: (0, 0)),
        ],
        out_specs=pl.BlockSpec((_BN, nc), lambda i: (i, 0)),
        out_shape=jax.ShapeDtypeStruct((n, nc), jnp.float32),
    )(acc2, ys1, dinv, y1, w2, b2, g2, be2, wo1, bo1, wo2, bo2)


def kernel(x, edge_index, W1, b1, g1, be1, W2, b2, g2, be2, Wo1, bo1, Wo2, bo2):
    n = x.shape[0]
    e = edge_index.shape[1]
    nw = _NCORES * _NSUB
    steps = e // (nw * _CHUNK)
    ei4 = edge_index.reshape(2, nw, steps, _CHUNK).transpose(1, 2, 0, 3)
    degp = _deg_call(ei4, n)
    xs, dinv = _tc1_call(x, degp)
    acc1 = _agg_call(xs, ei4)
    y1, ys1 = _tc2_call(acc1, xs, dinv, x, W1, b1.reshape(1, -1),
                        g1.reshape(1, -1), be1.reshape(1, -1))
    acc2 = _agg_call(ys1, ei4)
    o = _tc3_call(acc2, ys1, dinv, y1, W2, b2.reshape(1, -1),
                  g2.reshape(1, -1), be2.reshape(1, -1), Wo1,
                  bo1.reshape(1, -1), Wo2, bo2.reshape(1, -1))
    return o


# speedup vs baseline: 1.0245x; 1.0245x over previous
import functools

import jax
import jax.numpy as jnp
from jax import lax
from jax.experimental import pallas as pl
from jax.experimental.pallas import tpu as pltpu
from jax.experimental.pallas import tpu_sc as plsc

_EPS = 1e-5
_NCORES = 2
_NSUB = 16
_CHUNK = 80
_ZROWS = 128
_NBUF = 4


def _pad_rows(n):
    q = _NSUB * _ZROWS
    return ((n + q - 1) // q) * q


def _fill_const(ref, rows, d, val):
    def body(i, carry):
        for l in range(d // 16):
            ref[i, pl.ds(l * 16, 16)] = jnp.full((16,), val, jnp.float32)
        return carry

    lax.fori_loop(0, rows, body, None)


def _deg_call(ei4, n):
    nw, steps, _, c = ei4.shape
    n_pad = _pad_rows(n)
    mesh = plsc.VectorSubcoreMesh(core_axis_name="c", subcore_axis_name="s")

    @functools.partial(
        pl.kernel,
        mesh=mesh,
        out_type=jax.ShapeDtypeStruct((nw, n_pad), jnp.float32),
        compiler_params=pltpu.CompilerParams(needs_layout_passes=False),
        scratch_types=[
            pltpu.VMEM((steps, 2, c), jnp.int32),
            pltpu.VMEM((n_pad,), jnp.float32),
        ],
    )
    def deg_k(ei_hbm, out_hbm, eidx, hist):
        cid = lax.axis_index("c")
        sid = lax.axis_index("s")
        wid = cid * _NSUB + sid

        def z(i, carry):
            hist[pl.ds(i * 16, 16)] = jnp.zeros((16,), jnp.float32)
            return carry

        lax.fori_loop(0, n_pad // 16, z, None)
        pltpu.sync_copy(ei_hbm.at[wid], eidx)
        ones16 = jnp.full((16,), 1.0, jnp.float32)

        def row(r, carry):
            for l in range(c // 16):
                v = eidx[r, 1, pl.ds(l * 16, 16)]
                plsc.addupdate_scatter(hist, [v], ones16)
            return carry

        lax.fori_loop(0, steps, row, None)
        pltpu.sync_copy(hist, out_hbm.at[wid])

    return deg_k(ei4)


_NIDX = 8


def _agg_call(hs, ei4):
    n, d = hs.shape
    nw, steps, _, c = ei4.shape
    epw = steps * c
    n_pad = _pad_rows(n)
    rpt = n_pad // _NSUB
    mesh = plsc.VectorSubcoreMesh(core_axis_name="c", subcore_axis_name="s")

    @functools.partial(
        pl.kernel,
        mesh=mesh,
        out_type=jax.ShapeDtypeStruct((_NCORES, n_pad, d), jnp.float32),
        scratch_types=(
            [pltpu.VMEM((c, d), jnp.float32)] * _NBUF
            + [pltpu.VMEM((2, c), jnp.int32)] * _NIDX
            + [pltpu.VMEM_SHARED((n_pad, d), jnp.float32),
               pltpu.SemaphoreType.DMA,
               pltpu.SemaphoreType.DMA,
               pltpu.SemaphoreType.DMA]
        ),
    )
    def agg_k(hs_hbm, ei_hbm, out_hbm, *refs):
        bufs = refs[:_NBUF]
        idxs = refs[_NBUF:_NBUF + _NIDX]
        acc_sh, isem, gsem, ssem = refs[_NBUF + _NIDX:]
        cid = lax.axis_index("c")
        sid = lax.axis_index("s")
        wid = cid * _NSUB + sid

        def idx_load(j, slot):
            pltpu.async_copy(ei_hbm.at[wid, j], idxs[slot], isem)

        def idx_drain(slot):
            pltpu.make_async_copy(ei_hbm.at[0, 0], idxs[slot], isem).wait()

        def g_drain(b):
            pltpu.make_async_copy(hs_hbm.at[pl.ds(0, c)], bufs[b],
                                  gsem).wait()

        def s_drain(b):
            pltpu.make_async_copy(hs_hbm.at[pl.ds(0, c)], bufs[b],
                                  ssem).wait()

        _fill_const(bufs[0], c, d, 0.0)
        t0 = sid * rpt
        nwb = rpt // c
        for k in range(nwb):
            pltpu.async_copy(bufs[0], acc_sh.at[pl.ds(t0 + k * c, c)], ssem)
        for j in range(4):
            idx_load(j, j)
        for k in range(nwb):
            s_drain(0)
        plsc.subcore_barrier()

        for j in range(2):
            idx_drain(j)
            pltpu.async_copy(hs_hbm.at[idxs[j].at[0]], bufs[j], gsem)

        def chunk_body(j, b, islot):
            static = isinstance(j, int)
            g_drain(b)
            pltpu.async_copy(bufs[b], acc_sh.at[idxs[islot].at[1]], ssem,
                             add=True)

            def drain_prev_scatter():
                s_drain((b + 2) % _NBUF)

            def next_gather():
                idx_drain((islot + 2) % _NIDX)
                pltpu.async_copy(hs_hbm.at[idxs[(islot + 2) % _NIDX].at[0]],
                                 bufs[(b + 2) % _NBUF], gsem)

            def next_idx():
                idx_load(j + 4, (islot + 4) % _NIDX)

            if static:
                if j >= 2:
                    drain_prev_scatter()
                if j + 2 < steps:
                    next_gather()
                if j + 4 < steps:
                    next_idx()
            else:
                pl.when(j >= 2)(drain_prev_scatter)
                pl.when(j + 2 < steps)(next_gather)
                pl.when(j + 4 < steps)(next_idx)

        def blk(jj, carry):
            j0 = jj * _NIDX
            for r in range(_NIDX):
                chunk_body(j0 + r, r % _NBUF, r)
            return carry

        lax.fori_loop(0, steps // _NIDX, blk, None)
        for r in range(steps % _NIDX):
            j = (steps // _NIDX) * _NIDX + r
            chunk_body(j, j % _NBUF, j % _NIDX)
        for r in range(2):
            s_drain(0)
        plsc.subcore_barrier()

        pltpu.async_copy(acc_sh.at[pl.ds(t0, c)], bufs[0], gsem)
        for k in range(nwb):
            b = k % 2
            g_drain(b)
            if k + 1 < nwb:
                pltpu.async_copy(acc_sh.at[pl.ds(t0 + (k + 1) * c, c)],
                                 bufs[(k + 1) % 2], gsem)
            pltpu.async_copy(bufs[b], out_hbm.at[cid, pl.ds(t0 + k * c, c)],
                             ssem)
            if k >= 1:
                s_drain(b)
        s_drain(0)

    return agg_k(hs, ei4)


_BN = 2000


def _tc1_call(x, w1, degp):
    n, d = x.shape
    h = w1.shape[1]
    nw = _NCORES * _NSUB
    bn = 2048

    def tc1(x_ref, w_ref, deg_ref, hs_ref, dinv_ref):
        cnt = jnp.dot(deg_ref[...].T, jnp.ones((nw, 1), jnp.float32),
                      preferred_element_type=jnp.float32)
        dinv = lax.rsqrt(cnt + 1.0)
        hv = jnp.dot(x_ref[...], w_ref[...], preferred_element_type=jnp.float32)
        hs_ref[...] = hv * dinv
        dinv_ref[...] = dinv

    return pl.pallas_call(
        tc1,
        grid=(pl.cdiv(n, bn),),
        in_specs=[
            pl.BlockSpec((bn, d), lambda i: (i, 0)),
            pl.BlockSpec((d, h), lambda i: (0, 0)),
            pl.BlockSpec((nw, bn), lambda i: (0, i)),
        ],
        out_specs=[
            pl.BlockSpec((bn, h), lambda i: (i, 0)),
            pl.BlockSpec((bn, 1), lambda i: (i, 0)),
        ],
        out_shape=[
            jax.ShapeDtypeStruct((n, h), jnp.float32),
            jax.ShapeDtypeStruct((n, 1), jnp.float32),
        ],
    )(x, w1, degp)


def _tc2_call(acc1, hs1, dinv, x, w2, b1, g1, be1):
    n, h = hs1.shape
    bn_scale = 1.0 / (1.0 + _EPS) ** 0.5

    def tc2(acc_ref, hs_ref, dinv_ref, x_ref, w_ref, b_ref, g_ref, be_ref,
            y_ref, hs2_ref):
        a = acc_ref[...]
        dinv = dinv_ref[...]
        pre = (a[0] + a[1] + hs_ref[...]) * dinv + b_ref[...]
        bn = pre * (g_ref[...] * bn_scale) + be_ref[...]
        y = jnp.maximum(bn, 0.0) + x_ref[...]
        y_ref[...] = y
        h2 = jnp.dot(y, w_ref[...], preferred_element_type=jnp.float32)
        hs2_ref[...] = h2 * dinv

    return pl.pallas_call(
        tc2,
        grid=(n // _BN,),
        in_specs=[
            pl.BlockSpec((_NCORES, _BN, h), lambda i: (0, i, 0)),
            pl.BlockSpec((_BN, h), lambda i: (i, 0)),
            pl.BlockSpec((_BN, 1), lambda i: (i, 0)),
            pl.BlockSpec((_BN, h), lambda i: (i, 0)),
            pl.BlockSpec((h, h), lambda i: (0, 0)),
            pl.BlockSpec((1, h), lambda i: (0, 0)),
            pl.BlockSpec((1, h), lambda i: (0, 0)),
            pl.BlockSpec((1, h), lambda i: (0, 0)),
        ],
        out_specs=[
            pl.BlockSpec((_BN, h), lambda i: (i, 0)),
            pl.BlockSpec((_BN, h), lambda i: (i, 0)),
        ],
        out_shape=[
            jax.ShapeDtypeStruct((n, h), jnp.float32),
            jax.ShapeDtypeStruct((n, h), jnp.float32),
        ],
    )(acc1, hs1, dinv, x, w2, b1, g1, be1)


def _tc3_call(acc2, hs2, dinv, y1, b2, g2, be2, wo1, bo1, wo2, bo2):
    n, h = hs2.shape
    hh = wo1.shape[1]
    nc = wo2.shape[1]
    bn_scale = 1.0 / (1.0 + _EPS) ** 0.5

    def tc3(acc_ref, hs_ref, dinv_ref, y1_ref, b_ref, g_ref, be_ref,
            wo1_ref, bo1_ref, wo2_ref, bo2_ref, o_ref):
        a = acc_ref[...]
        dinv = dinv_ref[...]
        pre = (a[0] + a[1] + hs_ref[...]) * dinv + b_ref[...]
        bn = pre * (g_ref[...] * bn_scale) + be_ref[...]
        y2 = jnp.maximum(bn, 0.0) + y1_ref[...]
        t = jnp.dot(y2, wo1_ref[...], preferred_element_type=jnp.float32)
        t = jnp.maximum(t + bo1_ref[...], 0.0)
        o = jnp.dot(t, wo2_ref[...], preferred_element_type=jnp.float32)
        o_ref[...] = o + bo2_ref[...]

    return pl.pallas_call(
        tc3,
        grid=(n // _BN,),
        in_specs=[
            pl.BlockSpec((_NCORES, _BN, h), lambda i: (0, i, 0)),
            pl.BlockSpec((_BN, h), lambda i: (i, 0)),
            pl.BlockSpec((_BN, 1), lambda i: (i, 0)),
            pl.BlockSpec((_BN, h), lambda i: (i, 0)),
            pl.BlockSpec((1, h), lambda i: (0, 0)),
            pl.BlockSpec((1, h), lambda i: (0, 0)),
            pl.BlockSpec((1, h), lambda i: (0, 0)),
            pl.BlockSpec((h, hh), lambda i: (0, 0)),
            pl.BlockSpec((1, hh), lambda i: (0, 0)),
            pl.BlockSpec((hh, nc), lambda i: (0, 0)),
            pl.BlockSpec((1, nc), lambda i: (0, 0)),
        ],
        out_specs=pl.BlockSpec((_BN, nc), lambda i: (i, 0)),
        out_shape=jax.ShapeDtypeStruct((n, nc), jnp.float32),
    )(acc2, hs2, dinv, y1, b2, g2, be2, wo1, bo1, wo2, bo2)


def kernel(x, edge_index, W1, b1, g1, be1, W2, b2, g2, be2, Wo1, bo1, Wo2, bo2):
    n = x.shape[0]
    e = edge_index.shape[1]
    nw = _NCORES * _NSUB
    steps = e // (nw * _CHUNK)
    ei4 = edge_index.reshape(2, nw, steps, _CHUNK).transpose(1, 2, 0, 3)
    degp = _deg_call(ei4, n)
    hs1, dinv = _tc1_call(x, W1, degp)
    acc1 = _agg_call(hs1, ei4)
    y1, hs2 = _tc2_call(acc1, hs1, dinv, x, W2, b1.reshape(1, -1),
                        g1.reshape(1, -1), be1.reshape(1, -1))
    acc2 = _agg_call(hs2, ei4)
    o = _tc3_call(acc2, hs2, dinv, y1, b2.reshape(1, -1),
                  g2.reshape(1, -1), be2.reshape(1, -1), Wo1,
                  bo1.reshape(1, -1), Wo2, bo2.reshape(1, -1))
    return o
